# Initial kernel scaffold; baseline (speedup 1.0000x reference)
import jax
import jax.numpy as jnp
from jax.experimental import pallas as pl


def kernel(x, trajectory, dcf):
    raise NotImplementedError("write your pallas kernel here")


# speedup vs baseline: 1.9008x; 1.9008x over previous
import math

import jax
import jax.numpy as jnp
from jax.experimental import pallas as pl


def _nufft_block_kernel(xr_ref, xi_ref, kx_ref, ky_ref, sdcf_ref, yr_ref, yi_ref):
    S = kx_ref.shape[1]
    N = xr_ref.shape[0]
    kxc = kx_ref[0]
    kyc = ky_ref[0]
    n = (jax.lax.broadcasted_iota(jnp.int32, (S, N), 1) - (N // 2)).astype(jnp.float32)
    tw = -2.0 * math.pi
    px = (tw * kxc) * n
    cx = jnp.cos(px)
    sx = jnp.sin(px)
    xr = xr_ref[...]
    xi = xi_ref[...]
    tr = (jnp.dot(cx, xr, preferred_element_type=jnp.float32)
          - jnp.dot(sx, xi, preferred_element_type=jnp.float32))
    ti = (jnp.dot(cx, xi, preferred_element_type=jnp.float32)
          + jnp.dot(sx, xr, preferred_element_type=jnp.float32))
    py = (tw * kyc) * n
    cy = jnp.cos(py)
    sy = jnp.sin(py)
    yr = jnp.sum(tr * cy - ti * sy, axis=1)
    yi = jnp.sum(tr * sy + ti * cy, axis=1)
    w = sdcf_ref[0, 0, :]
    yr_ref[0, 0, :] = yr * w
    yi_ref[0, 0, :] = yi * w


def kernel(x, trajectory, dcf):
    N = x.shape[0]
    K = trajectory.shape[1]
    S = 2048 if K % 2048 == 0 else K
    nblk = K // S
    xr = x[..., 0]
    xi = x[..., 1]
    kx = trajectory[0].reshape(nblk, S, 1)
    ky = trajectory[1].reshape(nblk, S, 1)
    sdcf = jnp.sqrt(dcf).reshape(nblk, 1, S)
    yr, yi = pl.pallas_call(
        _nufft_block_kernel,
        grid=(nblk,),
        in_specs=[
            pl.BlockSpec((N, N), lambda b: (0, 0)),
            pl.BlockSpec((N, N), lambda b: (0, 0)),
            pl.BlockSpec((1, S, 1), lambda b: (b, 0, 0)),
            pl.BlockSpec((1, S, 1), lambda b: (b, 0, 0)),
            pl.BlockSpec((1, 1, S), lambda b: (b, 0, 0)),
        ],
        out_specs=[
            pl.BlockSpec((1, 1, S), lambda b: (b, 0, 0)),
            pl.BlockSpec((1, 1, S), lambda b: (b, 0, 0)),
        ],
        out_shape=[
            jax.ShapeDtypeStruct((nblk, 1, S), jnp.float32),
            jax.ShapeDtypeStruct((nblk, 1, S), jnp.float32),
        ],
    )(xr, xi, kx, ky, sdcf)
    return jnp.stack([yr.reshape(K), yi.reshape(K)], axis=-1)


# speedup vs baseline: 11.6092x; 6.1076x over previous
import math

import jax
import jax.numpy as jnp
from jax.experimental import pallas as pl


def _cmul(ar, ai, br, bi):
    return ar * br - ai * bi, ar * bi + ai * br


def _build_powers(dr, di, e0r, e0i, nrows):
    er, ei = e0r, e0i
    sr, si = dr, di
    rows = 1
    while rows < nrows:
        nr, ni = _cmul(er, ei, sr, si)
        er = jnp.concatenate([er, nr], axis=0)
        ei = jnp.concatenate([ei, ni], axis=0)
        if 2 * rows < nrows:
            sr, si = _cmul(sr, si, sr, si)
        rows *= 2
    return er, ei


def _nufft_block_kernel(xrt_ref, xit_ref, kx_ref, ky_ref, sdcf_ref, yr_ref, yi_ref):
    N = xrt_ref.shape[0]
    S = kx_ref.shape[2]
    half = N // 2
    tw = -2.0 * math.pi
    ax = tw * kx_ref[0]
    ay = tw * ky_ref[0]
    a2 = jnp.concatenate([ax, ay], axis=0)
    c2 = jnp.cos(a2)
    s2 = jnp.sin(a2)
    dxr, dyr = c2[0:1], c2[1:2]
    dxi, dyi = s2[0:1], s2[1:2]

    hxr, hxi, hyr, hyi = dxr, dxi, dyr, dyi
    m = half
    while m > 1:
        hxr, hxi = _cmul(hxr, hxi, hxr, hxi)
        hyr, hyi = _cmul(hyr, hyi, hyr, hyi)
        m //= 2
    w = sdcf_ref[0]
    exr, exi = _build_powers(dxr, dxi, hxr, -hxi, N)
    eyr, eyi = _build_powers(dyr, dyi, hyr * w, -hyi * w, N)

    xrt = xrt_ref[...]
    xit = xit_ref[...]
    trt = (jnp.dot(xrt, exr, preferred_element_type=jnp.float32)
           - jnp.dot(xit, exi, preferred_element_type=jnp.float32))
    tit = (jnp.dot(xrt, exi, preferred_element_type=jnp.float32)
           + jnp.dot(xit, exr, preferred_element_type=jnp.float32))
    yr_ref[0, 0, :] = jnp.sum(trt * eyr - tit * eyi, axis=0)
    yi_ref[0, 0, :] = jnp.sum(trt * eyi + tit * eyr, axis=0)


def kernel(x, trajectory, dcf):
    N = x.shape[0]
    K = trajectory.shape[1]
    S = 2048 if K % 2048 == 0 else K
    nblk = K // S
    xrt = x[..., 0].T
    xit = x[..., 1].T
    kx = trajectory[0].reshape(nblk, 1, S)
    ky = trajectory[1].reshape(nblk, 1, S)
    sdcf = jnp.sqrt(dcf).reshape(nblk, 1, S)
    yr, yi = pl.pallas_call(
        _nufft_block_kernel,
        grid=(nblk,),
        in_specs=[
            pl.BlockSpec((N, N), lambda b: (0, 0)),
            pl.BlockSpec((N, N), lambda b: (0, 0)),
            pl.BlockSpec((1, 1, S), lambda b: (b, 0, 0)),
            pl.BlockSpec((1, 1, S), lambda b: (b, 0, 0)),
            pl.BlockSpec((1, 1, S), lambda b: (b, 0, 0)),
        ],
        out_specs=[
            pl.BlockSpec((1, 1, S), lambda b: (b, 0, 0)),
            pl.BlockSpec((1, 1, S), lambda b: (b, 0, 0)),
        ],
        out_shape=[
            jax.ShapeDtypeStruct((nblk, 1, S), jnp.float32),
            jax.ShapeDtypeStruct((nblk, 1, S), jnp.float32),
        ],
    )(xrt, xit, kx, ky, sdcf)
    return jnp.stack([yr.reshape(K), yi.reshape(K)], axis=-1)


# speedup vs baseline: 12.3566x; 1.0644x over previous
import math

import jax
import jax.numpy as jnp
from jax.experimental import pallas as pl


def _cmul(ar, ai, br, bi):
    return ar * br - ai * bi, ar * bi + ai * br


def _build_powers(dr, di, e0r, e0i, nrows):
    er, ei = e0r, e0i
    sr, si = dr, di
    rows = 1
    while rows < nrows:
        take = min(rows, nrows - rows)
        nr, ni = _cmul(er[:take], ei[:take], sr, si)
        er = jnp.concatenate([er, nr], axis=0)
        ei = jnp.concatenate([ei, ni], axis=0)
        if 2 * rows < nrows:
            sr, si = _cmul(sr, si, sr, si)
        rows += take
    return er, ei


def _nufft_block_kernel(fpur_ref, fpui_ref, fpvr_ref, fpvi_ref,
                        fmur_ref, fmui_ref, fmvr_ref, fmvi_ref,
                        kx_ref, ky_ref, sdcf_ref, yr_ref, yi_ref):
    G = fpur_ref.shape[0]
    tw = -2.0 * math.pi
    ax = tw * kx_ref[0]
    ay = tw * ky_ref[0]
    a2 = jnp.concatenate([ax, ay], axis=0)
    c2 = jnp.cos(a2)
    s2 = jnp.sin(a2)
    dxr, dyr = c2[0:1], c2[1:2]
    dxi, dyi = s2[0:1], s2[1:2]

    one = jnp.ones_like(ax)
    zero = jnp.zeros_like(ax)
    w = sdcf_ref[0]
    cx, sx = _build_powers(dxr, dxi, one, zero, G)
    cy, sy = _build_powers(dyr, dyi, w, zero, G)

    def dot(a_ref, b):
        return jnp.dot(a_ref[...], b, preferred_element_type=jnp.float32)

    ur = dot(fpur_ref, cx) - dot(fpvi_ref, sx)
    ui = dot(fpui_ref, cx) + dot(fpvr_ref, sx)
    vr = dot(fmur_ref, cx) - dot(fmvi_ref, sx)
    vi = dot(fmui_ref, cx) + dot(fmvr_ref, sx)
    yr_ref[0, 0, :] = jnp.sum(ur * cy - vi * sy, axis=0)
    yi_ref[0, 0, :] = jnp.sum(ui * cy + vr * sy, axis=0)


def _fold_weights(x):
    N = x.shape[0]
    G = N // 2 + 8
    xrt = x[..., 0].T
    xit = x[..., 1].T

    def cfold(m):
        a = m[:, N // 2:]
        b = m[:, N // 2:0:-1]
        zp = jnp.zeros((m.shape[0], G - N // 2 - 1), jnp.float32)
        plus = jnp.concatenate([a + b, m[:, 0:1], zp], axis=1)
        minus = jnp.concatenate([a - b, -m[:, 0:1], zp], axis=1)
        return plus, minus

    def rfold(m):
        a = m[N // 2:, :]
        b = m[N // 2:0:-1, :]
        zp = jnp.zeros((G - N // 2 - 1, G), jnp.float32)
        plus = jnp.concatenate([a + b, m[0:1, :], zp], axis=0)
        minus = jnp.concatenate([a - b, -m[0:1, :], zp], axis=0)
        return plus, minus

    ur, vr = cfold(xrt)
    ui, vi = cfold(xit)
    fpur, fmur = rfold(ur)
    fpui, fmui = rfold(ui)
    fpvr, fmvr = rfold(vr)
    fpvi, fmvi = rfold(vi)
    half_col = jnp.ones((1, G), jnp.float32).at[0, 0].set(0.5)
    half_row = half_col.T
    fp = [f * half_col * half_row for f in (fpur, fpui, fpvr, fpvi)]
    fm = [f * half_col for f in (fmur, fmui, fmvr, fmvi)]
    return fp + fm, G


def kernel(x, trajectory, dcf):
    K = trajectory.shape[1]
    S = 2048 if K % 2048 == 0 else K
    nblk = K // S
    fmats, G = _fold_weights(x)
    kx = trajectory[0].reshape(nblk, 1, S)
    ky = trajectory[1].reshape(nblk, 1, S)
    sdcf = jnp.sqrt(dcf).reshape(nblk, 1, S)
    fspec = pl.BlockSpec((G, G), lambda b: (0, 0))
    rspec = pl.BlockSpec((1, 1, S), lambda b: (b, 0, 0))
    yr, yi = pl.pallas_call(
        _nufft_block_kernel,
        grid=(nblk,),
        in_specs=[fspec] * 8 + [rspec] * 3,
        out_specs=[rspec, rspec],
        out_shape=[
            jax.ShapeDtypeStruct((nblk, 1, S), jnp.float32),
            jax.ShapeDtypeStruct((nblk, 1, S), jnp.float32),
        ],
    )(*fmats, kx, ky, sdcf)
    return jnp.stack([yr.reshape(K), yi.reshape(K)], axis=-1)


# speedup vs baseline: 12.9425x; 1.0474x over previous
import math

import jax
import jax.numpy as jnp
from jax.experimental import pallas as pl


def _cmul(ar, ai, br, bi):
    return ar * br - ai * bi, ar * bi + ai * br


def _build_powers(dr, di, e0r, e0i, nrows):
    er, ei = e0r, e0i
    sr, si = dr, di
    rows = 1
    while rows < nrows:
        take = min(rows, nrows - rows)
        nr, ni = _cmul(er[:take], ei[:take], sr, si)
        er = jnp.concatenate([er, nr], axis=0)
        ei = jnp.concatenate([ei, ni], axis=0)
        if 2 * rows < nrows:
            sr, si = _cmul(sr, si, sr, si)
        rows += take
    return er, ei


def _nufft_block_kernel(fpur_ref, fpui_ref, fpvr_ref, fpvi_ref,
                        fmur_ref, fmui_ref, fmvr_ref, fmvi_ref,
                        kx_ref, ky_ref, sdcf_ref, yr_ref, yi_ref):
    G = fpur_ref.shape[0]
    tw = -2.0 * math.pi
    ax = tw * kx_ref[0]
    ay = tw * ky_ref[0]
    a2 = jnp.concatenate([ax, ay], axis=0)
    c2 = jnp.cos(a2)
    s2 = jnp.sin(a2)
    dxr, dyr = c2[0:1], c2[1:2]
    dxi, dyi = s2[0:1], s2[1:2]

    one = jnp.ones_like(ax)
    zero = jnp.zeros_like(ax)
    w = sdcf_ref[0]
    cx, sx = _build_powers(dxr, dxi, one, zero, G)
    cy, sy = _build_powers(dyr, dyi, w, zero, G)

    def dot(a_ref, b):
        return jnp.dot(a_ref[...], b, preferred_element_type=jnp.float32)

    ur = dot(fpur_ref, cx) - dot(fpvi_ref, sx)
    ui = dot(fpui_ref, cx) + dot(fpvr_ref, sx)
    vr = dot(fmur_ref, cx) - dot(fmvi_ref, sx)
    vi = dot(fmui_ref, cx) + dot(fmvr_ref, sx)
    yr_ref[0, 0, :] = jnp.sum(ur * cy - vi * sy, axis=0)
    yi_ref[0, 0, :] = jnp.sum(ui * cy + vr * sy, axis=0)


def _fold_weights(x):
    N = x.shape[0]
    G = N // 2 + 8
    xrt = x[..., 0].T
    xit = x[..., 1].T

    def cfold(m):
        a = m[:, N // 2:]
        b = m[:, N // 2:0:-1]
        zp = jnp.zeros((m.shape[0], G - N // 2 - 1), jnp.float32)
        plus = jnp.concatenate([a + b, m[:, 0:1], zp], axis=1)
        minus = jnp.concatenate([a - b, -m[:, 0:1], zp], axis=1)
        return plus, minus

    def rfold(m):
        a = m[N // 2:, :]
        b = m[N // 2:0:-1, :]
        zp = jnp.zeros((G - N // 2 - 1, G), jnp.float32)
        plus = jnp.concatenate([a + b, m[0:1, :], zp], axis=0)
        minus = jnp.concatenate([a - b, -m[0:1, :], zp], axis=0)
        return plus, minus

    ur, vr = cfold(xrt)
    ui, vi = cfold(xit)
    fpur, fmur = rfold(ur)
    fpui, fmui = rfold(ui)
    fpvr, fmvr = rfold(vr)
    fpvi, fmvi = rfold(vi)
    half_col = jnp.ones((1, G), jnp.float32).at[0, 0].set(0.5)
    half_row = half_col.T
    fp = [f * half_col * half_row for f in (fpur, fpui, fpvr, fpvi)]
    fm = [f * half_col for f in (fmur, fmui, fmvr, fmvi)]
    return fp + fm, G


def kernel(x, trajectory, dcf):
    K = trajectory.shape[1]
    S = 4096 if K % 4096 == 0 else K
    nblk = K // S
    fmats, G = _fold_weights(x)
    kx = trajectory[0].reshape(nblk, 1, S)
    ky = trajectory[1].reshape(nblk, 1, S)
    sdcf = jnp.sqrt(dcf).reshape(nblk, 1, S)
    fspec = pl.BlockSpec((G, G), lambda b: (0, 0))
    rspec = pl.BlockSpec((1, 1, S), lambda b: (b, 0, 0))
    yr, yi = pl.pallas_call(
        _nufft_block_kernel,
        grid=(nblk,),
        in_specs=[fspec] * 8 + [rspec] * 3,
        out_specs=[rspec, rspec],
        out_shape=[
            jax.ShapeDtypeStruct((nblk, 1, S), jnp.float32),
            jax.ShapeDtypeStruct((nblk, 1, S), jnp.float32),
        ],
    )(*fmats, kx, ky, sdcf)
    return jnp.stack([yr.reshape(K), yi.reshape(K)], axis=-1)
